# Initial kernel scaffold; baseline (speedup 1.0000x reference)
import jax
import jax.numpy as jnp
from jax.experimental import pallas as pl


def kernel(x, edge_index, Ws, att_src, att_dst, bias):
    raise NotImplementedError("write your pallas kernel here")


# speedup vs baseline: 1.9620x; 1.9620x over previous
import functools
import jax
import jax.numpy as jnp
from jax import lax
from jax.experimental import pallas as pl
from jax.experimental.pallas import tpu as pltpu

N = 50000
E = 800000
D = 64
H = 8
C = 8
L = 8
LANES = 128

BN = 256
N_PAD = ((N + 8) + BN - 1) // BN * BN
BE = 2048
NB = (E + BE - 1) // BE
E_PAD = NB * BE


def _k1_body(x_ref, w_ref, asf_ref, adf_ref, t_ref):
    xb = x_ref[...]
    h = jnp.dot(xb, w_ref[...], preferred_element_type=jnp.float32)
    i = lax.broadcasted_iota(jnp.int32, (H * C, H), 0)
    j = lax.broadcasted_iota(jnp.int32, (H * C, H), 1)
    S = jnp.where(i // C == j, 1.0, 0.0).astype(jnp.float32)
    t_ref[:, :D] = h
    t_ref[:, D:D + H] = jnp.dot(h * asf_ref[...], S,
                                preferred_element_type=jnp.float32)
    t_ref[:, D + H:D + 2 * H] = jnp.dot(h * adf_ref[...], S,
                                        preferred_element_type=jnp.float32)
    t_ref[:, D + 2 * H:] = jnp.zeros((xb.shape[0], LANES - D - 2 * H),
                                     jnp.float32)


def _k2_body(src_ref, dst_ref, t_ref, acc_ref, rows_ref, adrow_ref, comb_ref):
    @pl.when(pl.program_id(0) == 0)
    def _():
        acc_ref[...] = jnp.zeros_like(acc_ref)

    def gather(e, _):
        s = src_ref[0, 0, e]
        d = dst_ref[0, 0, e]
        rows_ref[pl.ds(e, 1), :] = t_ref[pl.ds(s, 1), :]
        adrow_ref[pl.ds(e, 1), :] = t_ref[pl.ds(d, 1), D + H:D + 2 * H]
        return 0

    lax.fori_loop(0, BE, gather, 0)

    e = rows_ref[:, D:D + H] + adrow_ref[...]
    e = jnp.where(e >= 0, e, 0.2 * e)
    ex = jnp.exp(e)
    jj = lax.broadcasted_iota(jnp.int32, (H, H * C), 0)
    kk = lax.broadcasted_iota(jnp.int32, (H, H * C), 1)
    B = jnp.where(kk // C == jj, 1.0, 0.0).astype(jnp.float32)
    exb = jnp.dot(ex, B, preferred_element_type=jnp.float32)
    comb_ref[:, :D] = rows_ref[:, :D] * exb
    comb_ref[:, D:D + H] = ex
    comb_ref[:, D + H:] = jnp.zeros((BE, LANES - D - H), jnp.float32)

    def scatter(e, _):
        d = dst_ref[0, 0, e]
        acc_ref[pl.ds(d, 1), :] = acc_ref[pl.ds(d, 1), :] + comb_ref[pl.ds(e, 1), :]
        return 0

    lax.fori_loop(0, BE, scatter, 0)


def _k3_body(acc_ref, b_ref, out_ref):
    num = acc_ref[:, :D]
    den = acc_ref[:, D:D + H]
    jj = lax.broadcasted_iota(jnp.int32, (H, H * C), 0)
    kk = lax.broadcasted_iota(jnp.int32, (H, H * C), 1)
    B = jnp.where(kk // C == jj, 1.0, 0.0).astype(jnp.float32)
    denb = jnp.dot(den, B, preferred_element_type=jnp.float32) + 1e-16
    out_ref[...] = num / denb + b_ref[...]


@jax.jit
def kernel(x, edge_index, Ws, att_src, att_dst, bias):
    src = edge_index[0]
    dst = edge_index[1]
    pad = E_PAD - E
    src_p = jnp.concatenate([src, jnp.full((pad,), N, jnp.int32)]).reshape(NB, 1, BE)
    dst_p = jnp.concatenate([dst, jnp.full((pad,), N, jnp.int32)]).reshape(NB, 1, BE)

    asf = att_src.reshape(L, 1, H * C)
    adf = att_dst.reshape(L, 1, H * C)

    k1 = pl.pallas_call(
        _k1_body,
        grid=(N_PAD // BN,),
        in_specs=[
            pl.BlockSpec((BN, D), lambda b: (b, 0)),
            pl.BlockSpec((D, H * C), lambda b: (0, 0)),
            pl.BlockSpec((1, H * C), lambda b: (0, 0)),
            pl.BlockSpec((1, H * C), lambda b: (0, 0)),
        ],
        out_specs=pl.BlockSpec((BN, LANES), lambda b: (b, 0)),
        out_shape=jax.ShapeDtypeStruct((N_PAD, LANES), jnp.float32),
    )

    k2 = pl.pallas_call(
        _k2_body,
        grid=(NB,),
        in_specs=[
            pl.BlockSpec((1, 1, BE), lambda b: (b, 0, 0), memory_space=pltpu.SMEM),
            pl.BlockSpec((1, 1, BE), lambda b: (b, 0, 0), memory_space=pltpu.SMEM),
            pl.BlockSpec((N_PAD, LANES), lambda b: (0, 0)),
        ],
        out_specs=pl.BlockSpec((N_PAD, LANES), lambda b: (0, 0)),
        out_shape=jax.ShapeDtypeStruct((N_PAD, LANES), jnp.float32),
        scratch_shapes=[
            pltpu.VMEM((BE, LANES), jnp.float32),
            pltpu.VMEM((BE, H), jnp.float32),
            pltpu.VMEM((BE, LANES), jnp.float32),
        ],
        compiler_params=pltpu.CompilerParams(
            dimension_semantics=("arbitrary",),
        ),
    )

    k3 = pl.pallas_call(
        _k3_body,
        grid=(N_PAD // BN,),
        in_specs=[
            pl.BlockSpec((BN, LANES), lambda b: (b, 0)),
            pl.BlockSpec((1, H * C), lambda b: (0, 0)),
        ],
        out_specs=pl.BlockSpec((BN, D), lambda b: (b, 0)),
        out_shape=jax.ShapeDtypeStruct((N_PAD, D), jnp.float32),
    )

    xp = jnp.zeros((N_PAD, D), jnp.float32).at[:N].set(x)
    for l in range(L):
        t = k1(xp, Ws[l], asf[l], adf[l])
        acc = k2(src_p, dst_p, t)
        xp = k3(acc, bias[l].reshape(1, H * C))
    return xp[:N]


# speedup vs baseline: 11.2827x; 5.7506x over previous
import functools
import jax
import jax.numpy as jnp
from jax import lax
from jax.experimental import pallas as pl
from jax.experimental.pallas import tpu as pltpu

N = 50000
E = 800000
D = 64
H = 8
C = 8
L = 8
LANES = 128

BN = 256
N_PAD = ((N + 8) + BN - 1) // BN * BN
BE = 2048
NB = (E + BE - 1) // BE
E_PAD = NB * BE


def _k1_body(x_ref, w_ref, asf_ref, adf_ref, t_ref):
    xb = x_ref[...]
    h = jnp.dot(xb, w_ref[...], preferred_element_type=jnp.float32)
    i = lax.broadcasted_iota(jnp.int32, (H * C, H), 0)
    j = lax.broadcasted_iota(jnp.int32, (H * C, H), 1)
    S = jnp.where(i // C == j, 1.0, 0.0).astype(jnp.float32)
    t_ref[:, :D] = h
    t_ref[:, D:D + H] = jnp.dot(h * asf_ref[...], S,
                                preferred_element_type=jnp.float32)
    t_ref[:, D + H:D + 2 * H] = jnp.dot(h * adf_ref[...], S,
                                        preferred_element_type=jnp.float32)
    t_ref[:, D + 2 * H:] = jnp.zeros((xb.shape[0], LANES - D - 2 * H),
                                     jnp.float32)


def _k2_body(src_ref, dst_ref, t_ref, acc_ref, rows_ref, adrow_ref, comb_ref):
    @pl.when(pl.program_id(0) == 0)
    def _():
        acc_ref[...] = jnp.zeros_like(acc_ref)

    def gather(e, _):
        s = src_ref[0, 0, e]
        d = dst_ref[0, 0, e]
        rows_ref[pl.ds(e, 1), :] = t_ref[pl.ds(s, 1), :]
        adrow_ref[pl.ds(e, 1), :] = t_ref[pl.ds(d, 1), D + H:D + 2 * H]
        return 0

    lax.fori_loop(0, BE, gather, 0, unroll=8)

    e = rows_ref[:, D:D + H] + adrow_ref[...]
    e = jnp.where(e >= 0, e, 0.2 * e)
    ex = jnp.exp(e)
    jj = lax.broadcasted_iota(jnp.int32, (H, H * C), 0)
    kk = lax.broadcasted_iota(jnp.int32, (H, H * C), 1)
    B = jnp.where(kk // C == jj, 1.0, 0.0).astype(jnp.float32)
    exb = jnp.dot(ex, B, preferred_element_type=jnp.float32)
    comb_ref[:, :D] = rows_ref[:, :D] * exb
    comb_ref[:, D:D + H] = ex
    comb_ref[:, D + H:] = jnp.zeros((BE, LANES - D - H), jnp.float32)

    def scatter(e, _):
        d = dst_ref[0, 0, e]
        acc_ref[pl.ds(d, 1), :] = acc_ref[pl.ds(d, 1), :] + comb_ref[pl.ds(e, 1), :]
        return 0

    lax.fori_loop(0, BE, scatter, 0, unroll=8)


def _k3_body(acc_ref, b_ref, out_ref):
    num = acc_ref[:, :D]
    den = acc_ref[:, D:D + H]
    jj = lax.broadcasted_iota(jnp.int32, (H, H * C), 0)
    kk = lax.broadcasted_iota(jnp.int32, (H, H * C), 1)
    B = jnp.where(kk // C == jj, 1.0, 0.0).astype(jnp.float32)
    denb = jnp.dot(den, B, preferred_element_type=jnp.float32) + 1e-16
    out_ref[...] = num / denb + b_ref[...]


@jax.jit
def kernel(x, edge_index, Ws, att_src, att_dst, bias):
    src = edge_index[0]
    dst = edge_index[1]
    pad = E_PAD - E
    src_p = jnp.concatenate([src, jnp.full((pad,), N, jnp.int32)]).reshape(NB, 1, BE)
    dst_p = jnp.concatenate([dst, jnp.full((pad,), N, jnp.int32)]).reshape(NB, 1, BE)

    asf = att_src.reshape(L, 1, H * C)
    adf = att_dst.reshape(L, 1, H * C)

    k1 = pl.pallas_call(
        _k1_body,
        grid=(N_PAD // BN,),
        in_specs=[
            pl.BlockSpec((BN, D), lambda b: (b, 0)),
            pl.BlockSpec((D, H * C), lambda b: (0, 0)),
            pl.BlockSpec((1, H * C), lambda b: (0, 0)),
            pl.BlockSpec((1, H * C), lambda b: (0, 0)),
        ],
        out_specs=pl.BlockSpec((BN, LANES), lambda b: (b, 0)),
        out_shape=jax.ShapeDtypeStruct((N_PAD, LANES), jnp.float32),
    )

    k2 = pl.pallas_call(
        _k2_body,
        grid=(NB,),
        in_specs=[
            pl.BlockSpec((1, 1, BE), lambda b: (b, 0, 0), memory_space=pltpu.SMEM),
            pl.BlockSpec((1, 1, BE), lambda b: (b, 0, 0), memory_space=pltpu.SMEM),
            pl.BlockSpec((N_PAD, LANES), lambda b: (0, 0)),
        ],
        out_specs=pl.BlockSpec((N_PAD, LANES), lambda b: (0, 0)),
        out_shape=jax.ShapeDtypeStruct((N_PAD, LANES), jnp.float32),
        scratch_shapes=[
            pltpu.VMEM((BE, LANES), jnp.float32),
            pltpu.VMEM((BE, H), jnp.float32),
            pltpu.VMEM((BE, LANES), jnp.float32),
        ],
        compiler_params=pltpu.CompilerParams(
            dimension_semantics=("arbitrary",),
        ),
    )

    k3 = pl.pallas_call(
        _k3_body,
        grid=(N_PAD // BN,),
        in_specs=[
            pl.BlockSpec((BN, LANES), lambda b: (b, 0)),
            pl.BlockSpec((1, H * C), lambda b: (0, 0)),
        ],
        out_specs=pl.BlockSpec((BN, D), lambda b: (b, 0)),
        out_shape=jax.ShapeDtypeStruct((N_PAD, D), jnp.float32),
    )

    xp = jnp.zeros((N_PAD, D), jnp.float32).at[:N].set(x)
    for l in range(L):
        t = k1(xp, Ws[l], asf[l], adf[l])
        acc = k2(src_p, dst_p, t)
        xp = k3(acc, bias[l].reshape(1, H * C))
    return xp[:N]


# speedup vs baseline: 17.1221x; 1.5176x over previous
import functools
import jax
import jax.numpy as jnp
from jax import lax
from jax.experimental import pallas as pl
from jax.experimental.pallas import tpu as pltpu

N = 50000
E = 800000
D = 64
H = 8
C = 8
L = 8
LANES = 128

BN = 256
N_PAD = ((N + 8) + BN - 1) // BN * BN
BE = 2048
NB = (E + BE - 1) // BE
E_PAD = NB * BE


def _k1_body(x_ref, w_ref, asf_ref, adf_ref, t_ref):
    xb = x_ref[...]
    h = jnp.dot(xb, w_ref[...], preferred_element_type=jnp.float32)
    i = lax.broadcasted_iota(jnp.int32, (H * C, H), 0)
    j = lax.broadcasted_iota(jnp.int32, (H * C, H), 1)
    S = jnp.where(i // C == j, 1.0, 0.0).astype(jnp.float32)
    t_ref[:, :D] = h
    t_ref[:, D:D + H] = jnp.dot(h * asf_ref[...], S,
                                preferred_element_type=jnp.float32)
    t_ref[:, D + H:D + 2 * H] = jnp.dot(h * adf_ref[...], S,
                                        preferred_element_type=jnp.float32)
    t_ref[:, D + 2 * H:] = jnp.zeros((xb.shape[0], LANES - D - 2 * H),
                                     jnp.float32)


def _k2_body(src_ref, dst_ref, t_ref, acc_ref, rows_ref, adrow_ref, comb_ref):
    @pl.when(pl.program_id(0) == 0)
    def _():
        acc_ref[...] = jnp.zeros_like(acc_ref)

    def gather(e, _):
        s = src_ref[0, 0, e]
        d = dst_ref[0, 0, e]
        rows_ref[pl.ds(e, 1), :] = t_ref[pl.ds(s, 1), :]
        adrow_ref[pl.ds(e, 1), :] = t_ref[pl.ds(d, 1), D + H:D + 2 * H]
        return 0

    lax.fori_loop(0, BE, gather, 0, unroll=16)

    e = rows_ref[:, D:D + H] + adrow_ref[...]
    e = jnp.where(e >= 0, e, 0.2 * e)
    ex = jnp.exp(e)
    jj = lax.broadcasted_iota(jnp.int32, (H, H * C), 0)
    kk = lax.broadcasted_iota(jnp.int32, (H, H * C), 1)
    B = jnp.where(kk // C == jj, 1.0, 0.0).astype(jnp.float32)
    exb = jnp.dot(ex, B, preferred_element_type=jnp.float32)
    comb_ref[:, :D] = rows_ref[:, :D] * exb
    comb_ref[:, D:D + H] = ex
    comb_ref[:, D + H:] = jnp.zeros((BE, LANES - D - H), jnp.float32)

    def scatter(e, _):
        d = dst_ref[0, 0, e]
        acc_ref[pl.ds(d, 1), :] = acc_ref[pl.ds(d, 1), :] + comb_ref[pl.ds(e, 1), :]
        return 0

    lax.fori_loop(0, BE, scatter, 0, unroll=16)


def _k3_body(acc_ref, b_ref, out_ref):
    num = acc_ref[:, :D]
    den = acc_ref[:, D:D + H]
    jj = lax.broadcasted_iota(jnp.int32, (H, H * C), 0)
    kk = lax.broadcasted_iota(jnp.int32, (H, H * C), 1)
    B = jnp.where(kk // C == jj, 1.0, 0.0).astype(jnp.float32)
    denb = jnp.dot(den, B, preferred_element_type=jnp.float32) + 1e-16
    out_ref[...] = num / denb + b_ref[...]


@jax.jit
def kernel(x, edge_index, Ws, att_src, att_dst, bias):
    src = edge_index[0]
    dst = edge_index[1]
    pad = E_PAD - E
    src_p = jnp.concatenate([src, jnp.full((pad,), N, jnp.int32)]).reshape(NB, 1, BE)
    dst_p = jnp.concatenate([dst, jnp.full((pad,), N, jnp.int32)]).reshape(NB, 1, BE)

    asf = att_src.reshape(L, 1, H * C)
    adf = att_dst.reshape(L, 1, H * C)

    k1 = pl.pallas_call(
        _k1_body,
        grid=(N_PAD // BN,),
        in_specs=[
            pl.BlockSpec((BN, D), lambda b: (b, 0)),
            pl.BlockSpec((D, H * C), lambda b: (0, 0)),
            pl.BlockSpec((1, H * C), lambda b: (0, 0)),
            pl.BlockSpec((1, H * C), lambda b: (0, 0)),
        ],
        out_specs=pl.BlockSpec((BN, LANES), lambda b: (b, 0)),
        out_shape=jax.ShapeDtypeStruct((N_PAD, LANES), jnp.float32),
    )

    k2 = pl.pallas_call(
        _k2_body,
        grid=(NB,),
        in_specs=[
            pl.BlockSpec((1, 1, BE), lambda b: (b, 0, 0), memory_space=pltpu.SMEM),
            pl.BlockSpec((1, 1, BE), lambda b: (b, 0, 0), memory_space=pltpu.SMEM),
            pl.BlockSpec((N_PAD, LANES), lambda b: (0, 0)),
        ],
        out_specs=pl.BlockSpec((N_PAD, LANES), lambda b: (0, 0)),
        out_shape=jax.ShapeDtypeStruct((N_PAD, LANES), jnp.float32),
        scratch_shapes=[
            pltpu.VMEM((BE, LANES), jnp.float32),
            pltpu.VMEM((BE, H), jnp.float32),
            pltpu.VMEM((BE, LANES), jnp.float32),
        ],
        compiler_params=pltpu.CompilerParams(
            dimension_semantics=("arbitrary",),
        ),
    )

    k3 = pl.pallas_call(
        _k3_body,
        grid=(N_PAD // BN,),
        in_specs=[
            pl.BlockSpec((BN, LANES), lambda b: (b, 0)),
            pl.BlockSpec((1, H * C), lambda b: (0, 0)),
        ],
        out_specs=pl.BlockSpec((BN, D), lambda b: (b, 0)),
        out_shape=jax.ShapeDtypeStruct((N_PAD, D), jnp.float32),
    )

    xp = jnp.zeros((N_PAD, D), jnp.float32).at[:N].set(x)
    for l in range(L):
        t = k1(xp, Ws[l], asf[l], adf[l])
        acc = k2(src_p, dst_p, t)
        xp = k3(acc, bias[l].reshape(1, H * C))
    return xp[:N]
